# baseline (device time: 36124 ns/iter reference)
import jax
import jax.numpy as jnp
from jax import lax
from jax.experimental import pallas as pl
from jax.experimental.pallas import tpu as pltpu

K = 1024
H = 512
D = 1024


def kernel(partial, gamma):
    p = partial.reshape(2 * K, D)
    g = gamma.reshape(1, D)

    def body(p_ref, g_ref, out_ref, send_buf, recv_direct, recv_fwd,
             send_sems, recv_sems):
        my_x = lax.axis_index("x")
        my_y = lax.axis_index("y")
        other_x = 1 - my_x
        other_y = 1 - my_y

        barrier = pltpu.get_barrier_semaphore()
        pl.semaphore_signal(barrier, inc=1, device_id=(my_x, other_y),
                            device_id_type=pl.DeviceIdType.MESH)
        pl.semaphore_signal(barrier, inc=1, device_id=(other_x, my_y),
                            device_id_type=pl.DeviceIdType.MESH)
        pl.semaphore_wait(barrier, 2)

        send_row0 = other_y * K + my_x * H
        send_buf[...] = p_ref[pl.ds(send_row0, H), :].astype(jnp.bfloat16)

        rdma_y = pltpu.make_async_remote_copy(
            src_ref=send_buf, dst_ref=recv_direct,
            send_sem=send_sems.at[0], recv_sem=recv_sems.at[0],
            device_id=(my_x, other_y), device_id_type=pl.DeviceIdType.MESH)
        rdma_y.start()
        rdma_y.wait()

        rdma_x = pltpu.make_async_remote_copy(
            src_ref=recv_direct, dst_ref=recv_fwd,
            send_sem=send_sems.at[1], recv_sem=recv_sems.at[1],
            device_id=(other_x, my_y), device_id_type=pl.DeviceIdType.MESH)
        rdma_x.start()
        rdma_x.wait()

        base = my_y * K
        off_d = my_x * H
        off_f = other_x * H
        out_ref[pl.ds(off_d, H), :] = (
            p_ref[pl.ds(base + off_d, H), :]
            + recv_direct[...].astype(jnp.float32))
        out_ref[pl.ds(off_f, H), :] = (
            p_ref[pl.ds(base + off_f, H), :]
            + recv_fwd[...].astype(jnp.float32))

        y = out_ref[...]
        rms = jnp.sqrt(jnp.mean(y * y, axis=-1, keepdims=True) + 1e-6)
        out_ref[...] = y / rms * g_ref[...]

    return pl.pallas_call(
        body,
        out_shape=jax.ShapeDtypeStruct((K, D), jnp.float32),
        in_specs=[pl.BlockSpec(memory_space=pltpu.VMEM),
                  pl.BlockSpec(memory_space=pltpu.VMEM)],
        out_specs=pl.BlockSpec(memory_space=pltpu.VMEM),
        scratch_shapes=[
            pltpu.VMEM((H, D), jnp.bfloat16),
            pltpu.VMEM((H, D), jnp.bfloat16),
            pltpu.VMEM((H, D), jnp.bfloat16),
            pltpu.SemaphoreType.DMA((2,)),
            pltpu.SemaphoreType.DMA((2,)),
        ],
        compiler_params=pltpu.CompilerParams(collective_id=0),
    )(p, g)


# device time: 25847 ns/iter; 1.3976x vs baseline; 1.3976x over previous
import jax
import jax.numpy as jnp
from jax import lax
from jax.experimental import pallas as pl
from jax.experimental.pallas import tpu as pltpu

K = 1024
H = 512
D = 1024
C = 8
CH = H // C


def kernel(partial, gamma):
    p = partial.reshape(2 * K, D)
    g = gamma.reshape(1, D)

    def body(p_ref, g_ref, out_ref, send_buf, recv_direct, recv_fwd,
             y_send_sems, y_recv_sems, x_send_sems, x_recv_sems):
        my_x = lax.axis_index("x")
        my_y = lax.axis_index("y")
        other_x = 1 - my_x
        other_y = 1 - my_y

        barrier = pltpu.get_barrier_semaphore()
        pl.semaphore_signal(barrier, inc=1, device_id=(my_x, other_y),
                            device_id_type=pl.DeviceIdType.MESH)
        pl.semaphore_signal(barrier, inc=1, device_id=(other_x, my_y),
                            device_id_type=pl.DeviceIdType.MESH)
        pl.semaphore_wait(barrier, 2)

        send_row0 = other_y * K + my_x * H
        base = my_y * K
        off_d = my_x * H
        off_f = other_x * H

        rdma_y = []
        for i in range(C):
            r = pl.ds(i * CH, CH)
            send_buf[r, :] = p_ref[pl.ds(send_row0 + i * CH, CH), :].astype(
                jnp.bfloat16)
            rdma = pltpu.make_async_remote_copy(
                src_ref=send_buf.at[r], dst_ref=recv_direct.at[r],
                send_sem=y_send_sems.at[i], recv_sem=y_recv_sems.at[i],
                device_id=(my_x, other_y),
                device_id_type=pl.DeviceIdType.MESH)
            rdma.start()
            rdma_y.append(rdma)

        rdma_x = []
        for i in range(C):
            r = pl.ds(i * CH, CH)
            rdma_y[i].wait_recv()
            rdma = pltpu.make_async_remote_copy(
                src_ref=recv_direct.at[r], dst_ref=recv_fwd.at[r],
                send_sem=x_send_sems.at[i], recv_sem=x_recv_sems.at[i],
                device_id=(other_x, my_y),
                device_id_type=pl.DeviceIdType.MESH)
            rdma.start()
            rdma_x.append(rdma)
            out_ref[pl.ds(off_d + i * CH, CH), :] = (
                p_ref[pl.ds(base + off_d + i * CH, CH), :]
                + recv_direct[r, :].astype(jnp.float32))

        for i in range(C):
            r = pl.ds(i * CH, CH)
            rdma_x[i].wait_recv()
            out_ref[pl.ds(off_f + i * CH, CH), :] = (
                p_ref[pl.ds(base + off_f + i * CH, CH), :]
                + recv_fwd[r, :].astype(jnp.float32))

        for i in range(C):
            rdma_y[i].wait_send()
            rdma_x[i].wait_send()

        y = out_ref[...]
        rms = jnp.sqrt(jnp.mean(y * y, axis=-1, keepdims=True) + 1e-6)
        out_ref[...] = y / rms * g_ref[...]

    return pl.pallas_call(
        body,
        out_shape=jax.ShapeDtypeStruct((K, D), jnp.float32),
        in_specs=[pl.BlockSpec(memory_space=pltpu.VMEM),
                  pl.BlockSpec(memory_space=pltpu.VMEM)],
        out_specs=pl.BlockSpec(memory_space=pltpu.VMEM),
        scratch_shapes=[
            pltpu.VMEM((H, D), jnp.bfloat16),
            pltpu.VMEM((H, D), jnp.bfloat16),
            pltpu.VMEM((H, D), jnp.bfloat16),
            pltpu.SemaphoreType.DMA((C,)),
            pltpu.SemaphoreType.DMA((C,)),
            pltpu.SemaphoreType.DMA((C,)),
            pltpu.SemaphoreType.DMA((C,)),
        ],
        compiler_params=pltpu.CompilerParams(collective_id=0),
    )(p, g)


# device time: 25193 ns/iter; 1.4339x vs baseline; 1.0260x over previous
import jax
import jax.numpy as jnp
from jax import lax
from jax.experimental import pallas as pl
from jax.experimental.pallas import tpu as pltpu

K = 1024
H = 512
D = 1024
C = 8
CH = H // C


def kernel(partial, gamma):
    g = gamma.reshape(1, D)

    def body(p_ref, g_ref, out_ref, send_buf, recv_direct, recv_fwd,
             y_send_sems, y_recv_sems, x_send_sems, x_recv_sems):
        my_x = lax.axis_index("x")
        my_y = lax.axis_index("y")
        other_x = 1 - my_x
        other_y = 1 - my_y

        barrier = pltpu.get_barrier_semaphore()
        pl.semaphore_signal(barrier, inc=1, device_id=(my_x, other_y),
                            device_id_type=pl.DeviceIdType.MESH)
        pl.semaphore_signal(barrier, inc=1, device_id=(other_x, my_y),
                            device_id_type=pl.DeviceIdType.MESH)
        pl.semaphore_wait(barrier, 2)

        send_row0 = other_y * K + my_x * H
        base = my_y * K
        off_d = my_x * H
        off_f = other_x * H

        def fold(recv_ref, i, off):
            r = pl.ds(i * CH, CH)
            yc = (p_ref[0, pl.ds(base + off + i * CH, CH), :]
                  + recv_ref[r, :].astype(jnp.float32))
            rms = jnp.sqrt(jnp.mean(yc * yc, axis=-1, keepdims=True) + 1e-6)
            out_ref[pl.ds(off + i * CH, CH), :] = yc / rms * g_ref[...]

        rdma_y = []
        for i in range(C):
            r = pl.ds(i * CH, CH)
            send_buf[r, :] = p_ref[0, pl.ds(send_row0 + i * CH, CH), :].astype(
                jnp.bfloat16)
            rdma = pltpu.make_async_remote_copy(
                src_ref=send_buf.at[r], dst_ref=recv_direct.at[r],
                send_sem=y_send_sems.at[i], recv_sem=y_recv_sems.at[i],
                device_id=(my_x, other_y),
                device_id_type=pl.DeviceIdType.MESH)
            rdma.start()
            rdma_y.append(rdma)

        rdma_x = []
        for i in range(C):
            r = pl.ds(i * CH, CH)
            rdma_y[i].wait_recv()
            rdma = pltpu.make_async_remote_copy(
                src_ref=recv_direct.at[r], dst_ref=recv_fwd.at[r],
                send_sem=x_send_sems.at[i], recv_sem=x_recv_sems.at[i],
                device_id=(other_x, my_y),
                device_id_type=pl.DeviceIdType.MESH)
            rdma.start()
            rdma_x.append(rdma)
            fold(recv_direct, i, off_d)

        for i in range(C):
            rdma_x[i].wait_recv()
            fold(recv_fwd, i, off_f)

        for i in range(C):
            rdma_y[i].wait_send()
            rdma_x[i].wait_send()

    return pl.pallas_call(
        body,
        out_shape=jax.ShapeDtypeStruct((K, D), jnp.float32),
        in_specs=[pl.BlockSpec(memory_space=pltpu.VMEM),
                  pl.BlockSpec(memory_space=pltpu.VMEM)],
        out_specs=pl.BlockSpec(memory_space=pltpu.VMEM),
        scratch_shapes=[
            pltpu.VMEM((H, D), jnp.bfloat16),
            pltpu.VMEM((H, D), jnp.bfloat16),
            pltpu.VMEM((H, D), jnp.bfloat16),
            pltpu.SemaphoreType.DMA((C,)),
            pltpu.SemaphoreType.DMA((C,)),
            pltpu.SemaphoreType.DMA((C,)),
            pltpu.SemaphoreType.DMA((C,)),
        ],
        compiler_params=pltpu.CompilerParams(collective_id=0),
    )(partial, g)
